# Initial kernel scaffold; baseline (speedup 1.0000x reference)
import jax
import jax.numpy as jnp
from jax.experimental import pallas as pl


def kernel(type1, type2, primary_color, secondary_color, shape, size, evolution_stage, habitat, legendary, mythical, type1_table, type2_table, primary_color_table, secondary_color_table, shape_table, size_table, evolution_stage_table, habitat_table, legendary_table, mythical_table):
    raise NotImplementedError("write your pallas kernel here")


# speedup vs baseline: 10.5884x; 10.5884x over previous
import functools

import jax
import jax.numpy as jnp
from jax import lax
from jax.experimental import pallas as pl
from jax.experimental.pallas import tpu as pltpu
from jax.experimental.pallas import tpu_sc as plsc

B = 16384
DIMS = (10, 10, 8, 8, 6, 6, 6, 6, 2, 2)
ROWS = (18, 19, 10, 11, 14, 6, 3, 9, 2, 2)
NF = len(DIMS)
D_OUT = sum(DIMS)

COL_OFF = []
_acc = 0
for _d in DIMS:
    COL_OFF.append(_acc)
    _acc += _d

TBL_BASE = []
_acc = 0
for _n, _d in zip(ROWS, DIMS):
    TBL_BASE.append(_acc)
    _acc += _n * _d
TBL_WORDS = _acc
TBL_PAD = (TBL_WORDS + 7) // 8 * 8

FIELD_OF_COL = []
for _f, _d in enumerate(DIMS):
    FIELD_OF_COL.extend([_f] * _d)

NC, NS, L = 2, 16, 16
NW = NC * NS
CHUNK = B // NW
NGRP = CHUNK // L

_mesh = plsc.VectorSubcoreMesh(core_axis_name="c", subcore_axis_name="s")


@functools.partial(
    pl.kernel,
    out_type=jax.ShapeDtypeStruct((B, D_OUT), jnp.float32),
    mesh=_mesh,
    compiler_params=pltpu.CompilerParams(needs_layout_passes=False),
    scratch_types=[
        pltpu.VMEM((NF, CHUNK), jnp.int32),
        pltpu.VMEM((TBL_PAD,), jnp.float32),
        pltpu.VMEM((CHUNK, D_OUT), jnp.float32),
    ],
)
def _embed_sc(i0, i1, i2, i3, i4, i5, i6, i7, i8, i9, tbl_hbm, out_hbm,
              stage_v, tbl_v, out_v):
    wid = lax.axis_index("s") * NC + lax.axis_index("c")
    base = wid * CHUNK

    pltpu.sync_copy(tbl_hbm, tbl_v)
    idx_refs = (i0, i1, i2, i3, i4, i5, i6, i7, i8, i9)
    for f in range(NF):
        pltpu.sync_copy(idx_refs[f].at[pl.ds(base, CHUNK)], stage_v.at[f])

    iota = lax.iota(jnp.int32, L)

    def body(g, carry):
        row0 = g * L
        rows16 = row0 + iota
        sidx = []
        for f in range(NF):
            raw = stage_v[f, pl.ds(row0, L)]
            sidx.append(raw * DIMS[f] + TBL_BASE[f])
        for c in range(D_OUT):
            f = FIELD_OF_COL[c]
            j = c - COL_OFF[f]
            vals = plsc.load_gather(tbl_v, [sidx[f] + j])
            cols = jnp.full((L,), c, jnp.int32)
            plsc.store_scatter(out_v, [rows16, cols], vals)
        return carry

    lax.fori_loop(0, NGRP, body, 0)
    pltpu.sync_copy(out_v, out_hbm.at[pl.ds(base, CHUNK)])


def kernel(type1, type2, primary_color, secondary_color, shape, size,
           evolution_stage, habitat, legendary, mythical,
           type1_table, type2_table, primary_color_table,
           secondary_color_table, shape_table, size_table,
           evolution_stage_table, habitat_table, legendary_table,
           mythical_table):
    idxs = [x.astype(jnp.int32) for x in
            (type1, type2, primary_color, secondary_color, shape, size,
             evolution_stage, habitat, legendary, mythical)]
    tables = (type1_table, type2_table, primary_color_table,
              secondary_color_table, shape_table, size_table,
              evolution_stage_table, habitat_table, legendary_table,
              mythical_table)
    tbl_flat = jnp.concatenate([t.reshape(-1) for t in tables])
    tbl_flat = jnp.pad(tbl_flat, (0, TBL_PAD - TBL_WORDS))
    return _embed_sc(*idxs, tbl_flat)
